# Initial kernel scaffold; baseline (speedup 1.0000x reference)
import jax
import jax.numpy as jnp
from jax.experimental import pallas as pl


def kernel(x_tensor, E):
    raise NotImplementedError("write your pallas kernel here")


# speedup vs baseline: 1.6751x; 1.6751x over previous
import functools

import jax
import jax.numpy as jnp
from jax import lax
from jax.experimental import pallas as pl
from jax.experimental.pallas import tpu as pltpu
from jax.experimental.pallas import tpu_sc as plsc

DIM = 64
NC = 2
NS = 16
NW = NC * NS
N = 16384 * 50
N_PER_W = N // NW
CHUNK = 128
K = N_PER_W // CHUNK

_mesh = plsc.VectorSubcoreMesh(core_axis_name="c", subcore_axis_name="s")


@functools.partial(
    pl.kernel,
    out_type=jax.ShapeDtypeStruct((N, DIM), jnp.float32),
    mesh=_mesh,
    scratch_types=[
        pltpu.VMEM((K, CHUNK), jnp.int32),
        pltpu.VMEM((CHUNK, DIM), jnp.float32),
        pltpu.SemaphoreType.DMA,
    ],
    compiler_params=pltpu.CompilerParams(use_tc_tiling_on_sc=False),
)
def _gather_kernel(idx_hbm, table_hbm, out_hbm, idx_v, rows_v, gsem):
    wid = lax.axis_index("s") * NC + lax.axis_index("c")
    base = wid * N_PER_W
    pltpu.sync_copy(idx_hbm.at[wid], idx_v)

    def body(j, carry):
        pltpu.async_copy(table_hbm.at[idx_v.at[j]], rows_v, gsem).wait()
        pltpu.sync_copy(rows_v, out_hbm.at[pl.ds(base + j * CHUNK, CHUNK)])
        return carry

    lax.fori_loop(0, K, body, 0)


def kernel(x_tensor, E):
    idx = x_tensor.reshape(NW, K, CHUNK)
    out = _gather_kernel(idx, E)
    return out.reshape(x_tensor.shape[0], x_tensor.shape[1], DIM)


# speedup vs baseline: 1.8672x; 1.1147x over previous
import functools

import jax
import jax.numpy as jnp
from jax import lax
from jax.experimental import pallas as pl
from jax.experimental.pallas import tpu as pltpu
from jax.experimental.pallas import tpu_sc as plsc

DIM = 64
NC = 2
NS = 16
NW = NC * NS
N = 16384 * 50
N_PER_W = N // NW
CHUNK = 128
K = N_PER_W // CHUNK
HALF = 4
G = K // HALF
P = G // 2

_mesh = plsc.VectorSubcoreMesh(core_axis_name="c", subcore_axis_name="s")


@functools.partial(
    pl.kernel,
    out_type=jax.ShapeDtypeStruct((N, DIM), jnp.float32),
    mesh=_mesh,
    scratch_types=[
        pltpu.VMEM((K, CHUNK), jnp.int32),
        [pltpu.VMEM((CHUNK, DIM), jnp.float32) for _ in range(2 * HALF)],
        pltpu.SemaphoreType.DMA,
        pltpu.SemaphoreType.DMA,
    ],
    compiler_params=pltpu.CompilerParams(use_tc_tiling_on_sc=False),
)
def _gather_kernel(idx_hbm, table_hbm, out_hbm, idx_v, rows, gsem, wsem):
    wid = lax.axis_index("s") * NC + lax.axis_index("c")
    base = wid * N_PER_W
    pltpu.sync_copy(idx_hbm.at[wid], idx_v)

    def start_gather(j, buf):
        pltpu.make_async_copy(table_hbm.at[idx_v.at[j]], rows[buf], gsem).start()

    def wait_gather(buf):
        pltpu.make_async_copy(table_hbm.at[idx_v.at[0]], rows[buf], gsem).wait()

    def start_write(j, buf):
        pltpu.make_async_copy(
            rows[buf], out_hbm.at[pl.ds(base + j * CHUNK, CHUNK)], wsem
        ).start()

    def wait_write(buf):
        pltpu.make_async_copy(
            rows[buf], out_hbm.at[pl.ds(base, CHUNK)], wsem
        ).wait()

    for bl in range(HALF):
        start_gather(bl, bl)

    def body(tt, carry):
        for p in (0, 1):
            g = 2 * tt + p
            if p == 0:
                @pl.when(tt > 0)
                def _():
                    for bl in range(HALF):
                        wait_write((1 - p) * HALF + bl)
            else:
                for bl in range(HALF):
                    wait_write((1 - p) * HALF + bl)
            if p == 0:
                for bl in range(HALF):
                    start_gather((g + 1) * HALF + bl, (1 - p) * HALF + bl)
            else:
                @pl.when(tt < P - 1)
                def _():
                    for bl in range(HALF):
                        start_gather((g + 1) * HALF + bl, (1 - p) * HALF + bl)
            for bl in range(HALF):
                wait_gather(p * HALF + bl)
            for bl in range(HALF):
                start_write(g * HALF + bl, p * HALF + bl)
        return carry

    lax.fori_loop(0, P, body, 0)

    for bl in range(HALF):
        wait_write(HALF + bl)


def kernel(x_tensor, E):
    idx = x_tensor.reshape(NW, K, CHUNK)
    out = _gather_kernel(idx, E)
    return out.reshape(x_tensor.shape[0], x_tensor.shape[1], DIM)


# speedup vs baseline: 1.8693x; 1.0011x over previous
import functools

import jax
import jax.numpy as jnp
from jax import lax
from jax.experimental import pallas as pl
from jax.experimental.pallas import tpu as pltpu
from jax.experimental.pallas import tpu_sc as plsc

DIM = 64
NC = 2
NS = 16
NW = NC * NS
N = 16384 * 50
N_PER_W = N // NW
CHUNK = 256
K = N_PER_W // CHUNK
NBUF = 4
T = K // NBUF

_mesh = plsc.VectorSubcoreMesh(core_axis_name="c", subcore_axis_name="s")


@functools.partial(
    pl.kernel,
    out_type=jax.ShapeDtypeStruct((N, DIM), jnp.float32),
    mesh=_mesh,
    scratch_types=[
        pltpu.VMEM((K, CHUNK), jnp.int32),
        [pltpu.VMEM((CHUNK, DIM), jnp.float32) for _ in range(NBUF)],
        pltpu.SemaphoreType.DMA,
        pltpu.SemaphoreType.DMA,
    ],
    compiler_params=pltpu.CompilerParams(use_tc_tiling_on_sc=False),
)
def _gather_kernel(idx_hbm, table_hbm, out_hbm, idx_v, rows, gsem, wsem):
    wid = lax.axis_index("s") * NC + lax.axis_index("c")
    base = wid * N_PER_W
    pltpu.sync_copy(idx_hbm.at[wid], idx_v)

    def start_gather(j, buf):
        pltpu.make_async_copy(table_hbm.at[idx_v.at[j]], rows[buf], gsem).start()

    def wait_gather(buf):
        pltpu.make_async_copy(table_hbm.at[idx_v.at[0]], rows[buf], gsem).wait()

    def start_write(j, buf):
        pltpu.make_async_copy(
            rows[buf], out_hbm.at[pl.ds(base + j * CHUNK, CHUNK)], wsem
        ).start()

    def wait_write(buf):
        pltpu.make_async_copy(
            rows[buf], out_hbm.at[pl.ds(base, CHUNK)], wsem
        ).wait()

    start_gather(0, 0)

    def body(t, carry):
        for p in range(NBUF):
            nxt = (p + 1) % NBUF
            if p == NBUF - 1:
                wait_write(nxt)
            else:
                @pl.when(t > 0)
                def _():
                    wait_write(nxt)
            if p == NBUF - 1:
                @pl.when(t < T - 1)
                def _():
                    start_gather(t * NBUF + p + 1, nxt)
            else:
                start_gather(t * NBUF + p + 1, nxt)
            wait_gather(p)
            start_write(t * NBUF + p, p)
        return carry

    lax.fori_loop(0, T, body, 0)

    for p in range(NBUF - 1):
        wait_write((1 + p) % NBUF)


def kernel(x_tensor, E):
    idx = x_tensor.reshape(NW, K, CHUNK)
    out = _gather_kernel(idx, E)
    return out.reshape(x_tensor.shape[0], x_tensor.shape[1], DIM)
